# baseline (speedup 1.0000x reference)
import jax
import jax.numpy as jnp
from jax.experimental import pallas as pl
from jax.experimental.pallas import tpu as pltpu

DIM = 768
HID = 768
E = 8
T = 2048
CT = 512


def _moe_body(xf_ref, xb_ref, rW_ref, rb_ref, W1_ref, b1_ref, W2_ref, b2_ref,
              out_ref, w_scratch):
    e = pl.program_id(0)

    @pl.when(e == 0)
    def _router():
        logits = jnp.dot(xf_ref[...], rW_ref[...],
                         preferred_element_type=jnp.float32)
        logits = logits + rb_ref[0]
        m = jnp.max(logits, axis=-1, keepdims=True)
        p = jnp.exp(logits - m)
        w_scratch[...] = p / jnp.sum(p, axis=-1, keepdims=True)

    first = e == 0
    for c in range(T // CT):
        sl = pl.ds(c * CT, CT)
        xs = xb_ref[sl, :]
        h = jnp.dot(xs, W1_ref[0], preferred_element_type=jnp.float32)
        h = h + b1_ref[0, 0]
        h = 0.5 * h * (1.0 + jax.lax.erf(h * 0.7071067811865476))
        eo = jnp.dot(h.astype(jnp.bfloat16), W2_ref[0],
                     preferred_element_type=jnp.float32)
        eo = eo + b2_ref[0, 0]
        lane = jax.lax.broadcasted_iota(jnp.int32, (CT, E), 1)
        w_e = jnp.sum(jnp.where(lane == e, w_scratch[sl, :], 0.0), axis=-1,
                      keepdims=True)
        contrib = w_e * eo
        out_ref[sl, :] = jnp.where(first, contrib, out_ref[sl, :] + contrib)


def kernel(x, rW, rb, W1, b1, W2, b2):
    B, Tx, D = x.shape
    x2 = x.reshape(Tx, D)
    xb = x2.astype(jnp.bfloat16)
    W1b = W1.astype(jnp.bfloat16)
    W2b = W2.astype(jnp.bfloat16)
    out = pl.pallas_call(
        _moe_body,
        grid=(E,),
        in_specs=[
            pl.BlockSpec((T, DIM), lambda e: (0, 0)),
            pl.BlockSpec((T, DIM), lambda e: (0, 0)),
            pl.BlockSpec((DIM, E), lambda e: (0, 0)),
            pl.BlockSpec((1, E), lambda e: (0, 0)),
            pl.BlockSpec((1, DIM, HID), lambda e: (e, 0, 0)),
            pl.BlockSpec((1, 1, HID), lambda e: (e, 0, 0)),
            pl.BlockSpec((1, HID, DIM), lambda e: (e, 0, 0)),
            pl.BlockSpec((1, 1, DIM), lambda e: (e, 0, 0)),
        ],
        out_specs=pl.BlockSpec((T, DIM), lambda e: (0, 0)),
        out_shape=jax.ShapeDtypeStruct((Tx, DIM), jnp.float32),
        scratch_shapes=[pltpu.VMEM((T, E), jnp.float32)],
        compiler_params=pltpu.CompilerParams(
            dimension_semantics=("arbitrary",),
        ),
    )(x2, xb, rW, rb.reshape(1, E), W1b, b1.reshape(E, 1, HID), W2b,
      b2.reshape(E, 1, DIM))
    return out.reshape(B, Tx, D)


# speedup vs baseline: 1.3833x; 1.3833x over previous
import jax
import jax.numpy as jnp
from jax.experimental import pallas as pl
from jax.experimental.pallas import tpu as pltpu

DIM = 768
HID = 768
E = 8
T = 2048
CT = 512


def _moe_body(x_ref, rW_ref, rb_ref, W1_ref, b1_ref, W2_ref, b2_ref,
              out_ref, w_scratch):
    e = pl.program_id(0)

    @pl.when(e == 0)
    def _router():
        logits = jnp.dot(x_ref[...], rW_ref[...],
                         preferred_element_type=jnp.float32)
        logits = logits + rb_ref[0]
        m = jnp.max(logits, axis=-1, keepdims=True)
        p = jnp.exp(logits - m)
        w_scratch[...] = p / jnp.sum(p, axis=-1, keepdims=True)

    first = e == 0
    for c in range(T // CT):
        sl = pl.ds(c * CT, CT)
        xs = x_ref[sl, :]
        h = jnp.dot(xs, W1_ref[0], preferred_element_type=jnp.float32)
        h = h + b1_ref[0, 0]
        h = 0.5 * h * (1.0 + jax.lax.erf(h * 0.7071067811865476))
        eo = jnp.dot(h, W2_ref[0], preferred_element_type=jnp.float32)
        eo = eo + b2_ref[0, 0]
        lane = jax.lax.broadcasted_iota(jnp.int32, (CT, E), 1)
        w_e = jnp.sum(jnp.where(lane == e, w_scratch[sl, :], 0.0), axis=-1,
                      keepdims=True)
        contrib = w_e * eo
        out_ref[sl, :] = jnp.where(first, contrib, out_ref[sl, :] + contrib)


def kernel(x, rW, rb, W1, b1, W2, b2):
    B, Tx, D = x.shape
    x2 = x.reshape(Tx, D)
    out = pl.pallas_call(
        _moe_body,
        grid=(E,),
        in_specs=[
            pl.BlockSpec((T, DIM), lambda e: (0, 0)),
            pl.BlockSpec((DIM, E), lambda e: (0, 0)),
            pl.BlockSpec((1, E), lambda e: (0, 0)),
            pl.BlockSpec((1, DIM, HID), lambda e: (e, 0, 0)),
            pl.BlockSpec((1, 1, HID), lambda e: (e, 0, 0)),
            pl.BlockSpec((1, HID, DIM), lambda e: (e, 0, 0)),
            pl.BlockSpec((1, 1, DIM), lambda e: (e, 0, 0)),
        ],
        out_specs=pl.BlockSpec((T, DIM), lambda e: (0, 0)),
        out_shape=jax.ShapeDtypeStruct((Tx, DIM), jnp.float32),
        scratch_shapes=[pltpu.VMEM((T, E), jnp.float32)],
        compiler_params=pltpu.CompilerParams(
            dimension_semantics=("arbitrary",),
        ),
    )(x2, rW, rb.reshape(1, E), W1, b1.reshape(E, 1, HID), W2,
      b2.reshape(E, 1, DIM))
    return out.reshape(B, Tx, D)
